# baseline (speedup 1.0000x reference)
import functools

import jax
import jax.numpy as jnp
from jax import lax
from jax.experimental import pallas as pl
from jax.experimental.pallas import tpu as pltpu
from jax.experimental.pallas import tpu_sc as plsc

_B = 4096
_M = 26
_T = 4
_L = 128
_E = 32
_EP = _E // 2
_EPS = 1e-5

_NC = 2
_NS = 16
_NW = _NC * _NS
_BC = _B // _NW


def _sc_gather(table, idx3):
    mesh = plsc.VectorSubcoreMesh(core_axis_name="c", subcore_axis_name="s")

    @functools.partial(
        pl.kernel,
        out_type=jax.ShapeDtypeStruct((_B, _M * _EP), jnp.int32),
        mesh=mesh,
        scratch_types=[
            pltpu.VMEM((_M, _T * _BC), jnp.int32),
            pltpu.VMEM((_T * _L, _EP), jnp.int32),
            pltpu.VMEM((_T * _L, _EP), jnp.int32),
            pltpu.VMEM((_BC, _EP), jnp.int32),
            pltpu.VMEM((_BC, _EP), jnp.int32),
            pltpu.SemaphoreType.DMA,
            pltpu.SemaphoreType.DMA,
            pltpu.SemaphoreType.DMA,
            pltpu.SemaphoreType.DMA,
        ],
        compiler_params=pltpu.CompilerParams(use_tc_tiling_on_sc=False,
                                             needs_layout_passes=False),
    )
    def k(table_hbm, idx_hbm, out_hbm, idx_v, tb0, tb1, eb0, eb1,
          gs0, gs1, ss0, ss1):
        wid = lax.axis_index("s") * _NC + lax.axis_index("c")
        base = wid * _BC
        pltpu.sync_copy(idx_hbm.at[:, wid], idx_v)

        tbufs = (tb0, tb1)
        ebufs = (eb0, eb1)
        gsems = (gs0, gs1)
        ssems = (ss0, ss1)

        def fire_slab(slot, m):
            pltpu.async_copy(table_hbm.at[pl.ds(m * (_T * _L), _T * _L)],
                             tbufs[slot], gsems[slot])

        def wait_slab(slot):
            pltpu.make_async_copy(table_hbm.at[pl.ds(0, _T * _L)],
                                  tbufs[slot], gsems[slot]).wait()

        def fire_store(slot, m):
            pltpu.async_copy(
                ebufs[slot],
                out_hbm.at[pl.ds(base, _BC), pl.ds(m * _EP, _EP)],
                ssems[slot])

        def wait_store(slot):
            pltpu.make_async_copy(
                ebufs[slot],
                out_hbm.at[pl.ds(base, _BC), pl.ds(0, _EP)],
                ssems[slot]).wait()

        def compute(slot, m):
            tb = tbufs[slot]
            eb = ebufs[slot]

            @plsc.parallel_loop(0, _BC // 16, 1)
            def grp(g, tb=tb, eb=eb, m=m):
                rows = g * 16 + lax.iota(jnp.int32, 16)
                r = [idx_v[m, pl.ds(t * _BC + g * 16, 16)]
                     for t in range(_T)]
                for p in range(_EP):
                    cp = jnp.full((16,), p, jnp.int32)
                    v0 = plsc.bitcast(plsc.load_gather(tb, [r[0], cp]),
                                      jnp.bfloat16)
                    v1 = plsc.bitcast(plsc.load_gather(tb, [r[1], cp]),
                                      jnp.bfloat16)
                    v2 = plsc.bitcast(plsc.load_gather(tb, [r[2], cp]),
                                      jnp.bfloat16)
                    v3 = plsc.bitcast(plsc.load_gather(tb, [r[3], cp]),
                                      jnp.bfloat16)
                    v = (v0 + v1) + (v2 + v3)
                    plsc.store_scatter(eb, [rows, cp],
                                       plsc.bitcast(v, jnp.int32))

        fire_slab(0, 0)
        fire_slab(1, 1)

        def step(j, carry):
            for slot in range(2):
                m = 2 * j + slot
                wait_slab(slot)

                @pl.when(j > 0)
                def _drain():
                    wait_store(slot)

                compute(slot, m)
                fire_slab(slot, m + 2)
                fire_store(slot, m)
            return carry

        lax.fori_loop(0, (_M - 2) // 2, step, 0)
        for slot in range(2):
            m = _M - 2 + slot
            wait_slab(slot)
            wait_store(slot)
            compute(slot, m)
            fire_store(slot, m)
        wait_store(0)
        wait_store(1)

    return k(table, idx3)


def _tc_head(emb, g2, be2, w2, bb):
    def body(z_ref, g_ref, be_ref, w_ref, bb_ref, o_ref, s_ref):
        z = z_ref[...].astype(jnp.float32)
        n = jnp.float32(_B)
        onesr = jnp.full((1, _B), 1.0, dtype=jnp.float32)
        s1 = jnp.dot(onesr, z, preferred_element_type=jnp.float32)
        s2 = jnp.dot(onesr, z * z, preferred_element_type=jnp.float32)
        mean = s1 / n
        var = s2 / n - mean * mean
        inv = lax.rsqrt(var + _EPS)
        gi = g_ref[...] * inv
        w = w_ref[...]
        coef = gi * w
        q = (be_ref[...] - mean * gi) * w
        rows = lax.broadcasted_iota(jnp.int32, (_M * _E, _M), 0)
        cols = lax.broadcasted_iota(jnp.int32, (_M * _E, _M), 1)
        sel = jnp.where(rows // _E == cols, jnp.float32(1.0),
                        jnp.float32(0.0))
        off = jnp.dot(q, sel, preferred_element_type=jnp.float32) + bb_ref[...]
        o = jnp.dot(z * coef, sel,
                    preferred_element_type=jnp.float32) + off
        o_ref[...] = o
        s_ref[...] = jnp.sum(o, axis=1, keepdims=True)

    return pl.pallas_call(
        body,
        out_shape=(jax.ShapeDtypeStruct((_B, _M), jnp.float32),
                   jax.ShapeDtypeStruct((_B, 1), jnp.float32)),
    )(emb, g2, be2, w2, bb)


def kernel(x, embed_w, bn_gamma, bn_beta, bout_w, bout_b):
    xi = x.astype(jnp.int32)
    col = jnp.arange(_M * _T, dtype=jnp.int32)
    offs = (col % _T) * _L
    flat = xi + offs[None, :]
    idx3 = (flat.T.reshape(_M, _T, _NW, _BC).swapaxes(1, 2)
            .reshape(_M, _NW, _T * _BC))
    table_bf = embed_w.reshape(_M * _T * _L, _E).astype(jnp.bfloat16)
    table = lax.bitcast_convert_type(
        table_bf.reshape(_M * _T * _L, _EP, 2), jnp.int32)

    emb_p = _sc_gather(table, idx3)
    emb = lax.bitcast_convert_type(
        emb_p, jnp.bfloat16).reshape(_B, _M * _E)

    o, sum_out = _tc_head(
        emb,
        bn_gamma.reshape(1, _M * _E),
        bn_beta.reshape(1, _M * _E),
        bout_w.reshape(1, _M * _E),
        bout_b.reshape(1, _M),
    )
    return (sum_out, o)


# speedup vs baseline: 1.7173x; 1.7173x over previous
import functools

import jax
import jax.numpy as jnp
from jax import lax
from jax.experimental import pallas as pl
from jax.experimental.pallas import tpu as pltpu
from jax.experimental.pallas import tpu_sc as plsc

_B = 4096
_M = 26
_T = 4
_L = 128
_E = 32
_EPS = 1e-5

_NC = 2
_NS = 16
_NW = _NC * _NS
_BC = _B // _NW


def _sc_gather(table, idx3):
    mesh = plsc.VectorSubcoreMesh(core_axis_name="c", subcore_axis_name="s")

    @functools.partial(
        pl.kernel,
        out_type=jax.ShapeDtypeStruct((_B, _M * _E), jnp.bfloat16),
        mesh=mesh,
        scratch_types=[
            pltpu.VMEM((_M, _T * _BC), jnp.int32),
            pltpu.VMEM((_T * _BC, _E), jnp.bfloat16),
            pltpu.VMEM((_T * _BC, _E), jnp.bfloat16),
            pltpu.VMEM((_T * _BC, _E), jnp.bfloat16),
            pltpu.VMEM((_T * _BC, _E), jnp.bfloat16),
            pltpu.VMEM((_BC, _E), jnp.bfloat16),
            pltpu.VMEM((_BC, _E), jnp.bfloat16),
            pltpu.VMEM((_BC, _E), jnp.bfloat16),
            pltpu.VMEM((_BC, _E), jnp.bfloat16),
            pltpu.SemaphoreType.DMA,
            pltpu.SemaphoreType.DMA,
            pltpu.SemaphoreType.DMA,
            pltpu.SemaphoreType.DMA,
            pltpu.SemaphoreType.DMA,
            pltpu.SemaphoreType.DMA,
            pltpu.SemaphoreType.DMA,
            pltpu.SemaphoreType.DMA,
        ],
        compiler_params=pltpu.CompilerParams(use_tc_tiling_on_sc=False),
    )
    def k(table_hbm, idx_hbm, out_hbm, idx_v, gb0, gb1, gb2, gb3,
          eb0, eb1, eb2, eb3, gs0, gs1, gs2, gs3, ss0, ss1, ss2, ss3):
        wid = lax.axis_index("s") * _NC + lax.axis_index("c")
        base = wid * _BC
        pltpu.sync_copy(idx_hbm.at[:, wid], idx_v)

        gbufs = (gb0, gb1, gb2, gb3)
        ebufs = (eb0, eb1, eb2, eb3)
        gsems = (gs0, gs1, gs2, gs3)
        ssems = (ss0, ss1, ss2, ss3)

        def fire_gather(slot, m):
            pltpu.async_copy(table_hbm.at[idx_v.at[m]], gbufs[slot],
                             gsems[slot])

        def wait_gather(slot):
            pltpu.make_async_copy(table_hbm.at[idx_v.at[0]], gbufs[slot],
                                  gsems[slot]).wait()

        def fire_store(slot, m):
            pltpu.async_copy(
                ebufs[slot],
                out_hbm.at[pl.ds(base, _BC), pl.ds(m * _E, _E)],
                ssems[slot])

        def wait_store(slot):
            pltpu.make_async_copy(
                ebufs[slot],
                out_hbm.at[pl.ds(base, _BC), pl.ds(0, _E)],
                ssems[slot]).wait()

        def compute(slot):
            gb = gbufs[slot]
            eb = ebufs[slot]

            @plsc.parallel_loop(0, _BC, 1, unroll=4)
            def tree_sum(i, gb=gb, eb=eb):
                eb[i, :] = ((gb[i, :] + gb[_BC + i, :])
                            + (gb[2 * _BC + i, :] + gb[3 * _BC + i, :]))

        _NB = 4
        for i in range(_NB):
            fire_gather(i, i)

        _TAIL = _M % _NB + _NB
        _LOOPM = _M - _TAIL

        def step(j, carry):
            for slot in range(_NB):
                m = _NB * j + slot
                wait_gather(slot)

                @pl.when(j > 0)
                def _drain():
                    wait_store(slot)

                compute(slot)
                fire_gather(slot, m + _NB)
                fire_store(slot, m)
            return carry

        lax.fori_loop(0, _LOOPM // _NB, step, 0)
        for m in range(_LOOPM, _M):
            slot = m % _NB
            wait_gather(slot)
            wait_store(slot)
            compute(slot)
            if m + _NB < _M:
                fire_gather(slot, m + _NB)
            fire_store(slot, m)
        for slot in range(_NB):
            wait_store(slot)

    return k(table, idx3)


def _tc_head(emb, g2, be2, w2, bb):
    def body(z_ref, g_ref, be_ref, w_ref, bb_ref, o_ref, s_ref):
        z = z_ref[...].astype(jnp.float32)
        n = jnp.float32(_B)
        onesr = jnp.full((1, _B), 1.0, dtype=jnp.float32)
        s1 = jnp.dot(onesr, z, preferred_element_type=jnp.float32)
        s2 = jnp.dot(onesr, z * z, preferred_element_type=jnp.float32)
        mean = s1 / n
        var = s2 / n - mean * mean
        inv = lax.rsqrt(var + _EPS)
        gi = g_ref[...] * inv
        w = w_ref[...]
        coef = gi * w
        q = (be_ref[...] - mean * gi) * w
        rows = lax.broadcasted_iota(jnp.int32, (_M * _E, _M), 0)
        cols = lax.broadcasted_iota(jnp.int32, (_M * _E, _M), 1)
        sel = jnp.where(rows // _E == cols, jnp.float32(1.0),
                        jnp.float32(0.0))
        off = jnp.dot(q, sel, preferred_element_type=jnp.float32) + bb_ref[...]
        o = jnp.dot(z * coef, sel,
                    preferred_element_type=jnp.float32) + off
        o_ref[...] = o
        s_ref[...] = jnp.sum(o, axis=1, keepdims=True)

    return pl.pallas_call(
        body,
        out_shape=(jax.ShapeDtypeStruct((_B, _M), jnp.float32),
                   jax.ShapeDtypeStruct((_B, 1), jnp.float32)),
    )(emb, g2, be2, w2, bb)


def kernel(x, embed_w, bn_gamma, bn_beta, bout_w, bout_b):
    xi = x.astype(jnp.int32)
    col = jnp.arange(_M * _T, dtype=jnp.int32)
    offs = (col // _T) * (_T * _L) + (col % _T) * _L
    flat = xi + offs[None, :]
    idx3 = (flat.T.reshape(_M, _T, _NW, _BC).swapaxes(1, 2)
            .reshape(_M, _NW, _T * _BC))
    table = embed_w.reshape(_M * _T * _L, _E).astype(jnp.bfloat16)

    emb = _sc_gather(table, idx3)

    o, sum_out = _tc_head(
        emb,
        bn_gamma.reshape(1, _M * _E),
        bn_beta.reshape(1, _M * _E),
        bout_w.reshape(1, _M * _E),
        bout_b.reshape(1, _M),
    )
    return (sum_out, o)
